# baseline (speedup 1.0000x reference)
import jax
import jax.numpy as jnp
from jax.experimental import pallas as pl
from jax.experimental.pallas import tpu as pltpu

_TOKENS = 16384
_DF = 512
_BLK = 2048
_NB = _TOKENS // _BLK
_NSTEPS = 2 * _NB
_LANES = 128
_EPS = 1e-6


def _body(wsch_ref, wact_ref, x_ref, seg_ref, o_ref, xs_ref, smin_ref, smax_ref):
    s = pl.program_id(0)
    lane = jax.lax.broadcasted_iota(jnp.int32, (_BLK, _LANES), 1)

    @pl.when(s < _NB)
    def _reduce():
        off = pl.multiple_of(s * _BLK, _BLK)
        seg = seg_ref[pl.ds(off, _BLK), :]
        mask = seg == lane
        xb = x_ref[...]
        xs_ref[pl.ds(off, _BLK), :] = xb
        rmin = jnp.min(xb, axis=1, keepdims=True)
        rmax = jnp.max(xb, axis=1, keepdims=True)
        pmin = jnp.min(jnp.where(mask, rmin, jnp.inf), axis=0, keepdims=True)
        pmax = jnp.max(jnp.where(mask, rmax, -jnp.inf), axis=0, keepdims=True)

        @pl.when(s == 0)
        def _init():
            smin_ref[0:1, :] = pmin
            smax_ref[0:1, :] = pmax

        @pl.when(s > 0)
        def _acc():
            smin_ref[0:1, :] = jnp.minimum(smin_ref[0:1, :], pmin)
            smax_ref[0:1, :] = jnp.maximum(smax_ref[0:1, :], pmax)

    @pl.when(wact_ref[s] == 1)
    def _normalize():
        b = wsch_ref[s]
        off = pl.multiple_of(b * _BLK, _BLK)
        segb = seg_ref[pl.ds(off, _BLK), :]
        maskb = segb == lane
        smin = smin_ref[0:1, :]
        sinv = 1.0 / (smax_ref[0:1, :] - smin + _EPS)
        m = jnp.sum(jnp.where(maskb, smin, 0.0), axis=1, keepdims=True)
        r = jnp.sum(jnp.where(maskb, sinv, 0.0), axis=1, keepdims=True)
        xv = xs_ref[pl.ds(off, _BLK), :]
        o_ref[...] = (xv - m) * r


def _write_schedule(segment_ids):
    ends = (jnp.arange(1, _NB + 1) * _BLK) - 1
    last_ids = segment_ids[ends]
    first_occ = jnp.sum(
        segment_ids[None, :] < last_ids[:, None], axis=1, dtype=jnp.int32
    )
    done = first_occ // _BLK
    dpad = jnp.concatenate(
        [done[: _NB - 1], jnp.full((_NSTEPS - _NB + 1,), _NB, jnp.int32)]
    )
    steps = jnp.arange(_NSTEPS, dtype=jnp.int32)
    prefix = jax.lax.associative_scan(jnp.minimum, dpad - steps)
    written = prefix + steps
    wsched = jnp.maximum(written - 1, 0)
    wactive = jnp.diff(written, prepend=jnp.int32(0)).astype(jnp.int32)
    return wsched, wactive


def kernel(x, segment_ids):
    seg2d = segment_ids.reshape(_TOKENS, 1)
    wsched, wactive = _write_schedule(segment_ids)
    grid_spec = pltpu.PrefetchScalarGridSpec(
        num_scalar_prefetch=2,
        grid=(_NSTEPS,),
        in_specs=[
            pl.BlockSpec((_BLK, _DF), lambda s, wsch, wact: (jnp.minimum(s, _NB - 1), 0)),
            pl.BlockSpec((_TOKENS, 1), lambda s, wsch, wact: (0, 0)),
        ],
        out_specs=pl.BlockSpec((_BLK, _DF), lambda s, wsch, wact: (wsch[s], 0)),
        scratch_shapes=[
            pltpu.VMEM((_TOKENS, _DF), jnp.float32),
            pltpu.VMEM((8, _LANES), jnp.float32),
            pltpu.VMEM((8, _LANES), jnp.float32),
        ],
    )
    return pl.pallas_call(
        _body,
        grid_spec=grid_spec,
        out_shape=jax.ShapeDtypeStruct((_TOKENS, _DF), jnp.float32),
    )(wsched, wactive, x, seg2d)


# speedup vs baseline: 1.1877x; 1.1877x over previous
import jax
import jax.numpy as jnp
from jax.experimental import pallas as pl
from jax.experimental.pallas import tpu as pltpu

_TOKENS = 16384
_DF = 512
_BLK = 2048
_NB = _TOKENS // _BLK
_LANES = 128
_EPS = 1e-6


def _body(x_ref, seg_ref, o_ref, xs_ref, smin_ref, smax_ref, smem_ref, sem):
    i = pl.program_id(0)
    lane = jax.lax.broadcasted_iota(jnp.int32, (_BLK, _LANES), 1)

    off_i = pl.multiple_of(i * _BLK, _BLK)
    seg = seg_ref[pl.ds(off_i, _BLK), :]
    mask = seg == lane

    xb = x_ref[...]
    xs_ref[pl.ds(off_i, _BLK), :] = xb

    rmin = jnp.min(xb, axis=1, keepdims=True)
    rmax = jnp.max(xb, axis=1, keepdims=True)
    pmin = jnp.min(jnp.where(mask, rmin, jnp.inf), axis=0, keepdims=True)
    pmax = jnp.max(jnp.where(mask, rmax, -jnp.inf), axis=0, keepdims=True)

    @pl.when(i == 0)
    def _init():
        smin_ref[0:1, :] = pmin
        smax_ref[0:1, :] = pmax

    @pl.when(i > 0)
    def _acc():
        smin_ref[0:1, :] = jnp.minimum(smin_ref[0:1, :], pmin)
        smax_ref[0:1, :] = jnp.maximum(smax_ref[0:1, :], pmax)

    last_id = jnp.max(seg)
    riota = jax.lax.broadcasted_iota(jnp.int32, (_BLK, 1), 0) + i * _BLK
    fo_in = jnp.min(jnp.where(seg == last_id, riota, _TOKENS))
    prev_f = smem_ref[1]
    prev_last = smem_ref[2]
    frontier = jnp.where(
        jnp.logical_and(i > 0, last_id == prev_last), prev_f, fo_in
    )
    smem_ref[1] = frontier
    smem_ref[2] = last_id

    f_eff = jnp.where(i == _NB - 1, _TOKENS, frontier)
    done_blocks = f_eff // _BLK
    written = jnp.where(i == 0, 0, smem_ref[0])

    def _write_block(b, carry):
        off = pl.multiple_of(b * _BLK, _BLK)
        segb = seg_ref[pl.ds(off, _BLK), :]
        maskb = segb == lane
        smin = smin_ref[0:1, :]
        sinv = 1.0 / (smax_ref[0:1, :] - smin + _EPS)
        m = jnp.sum(jnp.where(maskb, smin, 0.0), axis=1, keepdims=True)
        r = jnp.sum(jnp.where(maskb, sinv, 0.0), axis=1, keepdims=True)
        xv = xs_ref[pl.ds(off, _BLK), :]
        xs_ref[pl.ds(off, _BLK), :] = (xv - m) * r
        pltpu.make_async_copy(
            xs_ref.at[pl.ds(off, _BLK), :],
            o_ref.at[pl.ds(off, _BLK), :],
            sem,
        ).start()
        return carry

    jax.lax.fori_loop(written, done_blocks, _write_block, 0)
    smem_ref[0] = done_blocks

    @pl.when(i == _NB - 1)
    def _drain():
        def _wait(b, carry):
            off = pl.multiple_of(b * _BLK, _BLK)
            pltpu.make_async_copy(
                xs_ref.at[pl.ds(off, _BLK), :],
                o_ref.at[pl.ds(off, _BLK), :],
                sem,
            ).wait()
            return carry

        jax.lax.fori_loop(0, _NB, _wait, 0)


def kernel(x, segment_ids):
    seg2d = segment_ids.reshape(_TOKENS, 1)
    return pl.pallas_call(
        _body,
        grid=(_NB,),
        in_specs=[
            pl.BlockSpec((_BLK, _DF), lambda i: (i, 0)),
            pl.BlockSpec((_TOKENS, 1), lambda i: (0, 0)),
        ],
        out_specs=pl.BlockSpec(memory_space=pltpu.MemorySpace.HBM),
        out_shape=jax.ShapeDtypeStruct((_TOKENS, _DF), jnp.float32),
        scratch_shapes=[
            pltpu.VMEM((_TOKENS, _DF), jnp.float32),
            pltpu.VMEM((8, _LANES), jnp.float32),
            pltpu.VMEM((8, _LANES), jnp.float32),
            pltpu.SMEM((4,), jnp.int32),
            pltpu.SemaphoreType.DMA,
        ],
    )(x, seg2d)
